# Initial kernel scaffold; baseline (speedup 1.0000x reference)
import jax
import jax.numpy as jnp
from jax.experimental import pallas as pl


def kernel(x, edge_attr, edge_index, emb_table, phi_w0, phi_b0, phi_w1, phi_b1, phi_w2, phi_b2, phi_w3, phi_b3, gcn_w1, gcn_b1, gcn_w2, gcn_b2):
    raise NotImplementedError("write your pallas kernel here")


# speedup vs baseline: 1.1970x; 1.1970x over previous
import functools

import jax
import jax.numpy as jnp
from jax import lax
from jax.experimental import pallas as pl
from jax.experimental.pallas import tpu as pltpu


def _ceil_to(a, b):
    return (a + b - 1) // b * b


def _prep_body(x_ref, emb_ref, out_ref):
    xb = x_ref[...]
    et = xb[:, 1:2].astype(jnp.int32)
    e_row0 = emb_ref[0:1, :]
    e_row1 = emb_ref[1:2, :]
    emb_sel = jnp.where(et == 0, e_row0, e_row1)
    nf = jnp.concatenate([xb[:, 0:1], xb[:, 2:5]], axis=1)
    pad = jnp.zeros((xb.shape[0], 16 - 4 - emb_sel.shape[1]), jnp.float32)
    out_ref[...] = jnp.concatenate([nf, emb_sel, pad], axis=1)


def _prep_table(xp, emb_table, n_pad):
    bn = 1024
    return pl.pallas_call(
        _prep_body,
        grid=(n_pad // bn,),
        in_specs=[
            pl.BlockSpec((bn, 5), lambda i: (i, 0)),
            pl.BlockSpec((2, emb_table.shape[1]), lambda i: (0, 0)),
        ],
        out_specs=pl.BlockSpec((bn, 16), lambda i: (i, 0)),
        out_shape=jax.ShapeDtypeStruct((n_pad, 16), jnp.float32),
    )(xp, emb_table)


def _mlp_body(gcol, grow, ea, w0, w0e, b0, w1, b1, w2, b2, w3, b3, out):
    x32 = jnp.concatenate([gcol[...], grow[...]], axis=1)
    ea_b = ea[...]
    h = jnp.dot(x32, w0[...], preferred_element_type=jnp.float32)
    h = jnp.maximum(h + ea_b * w0e[...] + b0[...], 0.0)
    h = jnp.maximum(jnp.dot(h, w1[...], preferred_element_type=jnp.float32) + b1[...], 0.0)
    h = jnp.maximum(jnp.dot(h, w2[...], preferred_element_type=jnp.float32) + b2[...], 0.0)
    m = jnp.dot(h, w3[...], preferred_element_type=jnp.float32) + b3[...]
    d0 = w3.shape[1] - 32
    col_id = lax.broadcasted_iota(jnp.int32, (1, w3.shape[1]), 1)
    out[...] = m + jnp.where(col_id == d0, ea_b, 0.0)


def _mlp(gcol, grow, eap, w0cat, w0e, b0, w1, b1, w2, b2, w3a, b3a):
    e_pad = gcol.shape[0]
    hid = w1.shape[0]
    d0a = w3a.shape[1]
    be = 1024
    full = lambda a: pl.BlockSpec(a.shape, lambda i: tuple(0 for _ in a.shape))
    return pl.pallas_call(
        _mlp_body,
        grid=(e_pad // be,),
        in_specs=[
            pl.BlockSpec((be, 16), lambda i: (i, 0)),
            pl.BlockSpec((be, 16), lambda i: (i, 0)),
            pl.BlockSpec((be, 1), lambda i: (i, 0)),
            full(w0cat), full(w0e), full(b0),
            full(w1), full(b1), full(w2), full(b2),
            full(w3a), full(b3a),
        ],
        out_specs=pl.BlockSpec((be, d0a), lambda i: (i, 0)),
        out_shape=jax.ShapeDtypeStruct((e_pad, d0a), jnp.float32),
    )(gcol, grow, eap, w0cat, w0e, b0, w1, b1, w2, b2, w3a, b3a)


def _c_body(h0aug, w1g, g1_out, dinv_out):
    a = h0aug[...]
    d0 = a.shape[1] - 32
    deg = a[:, d0:d0 + 1]
    dinv = jnp.where(deg > 0, lax.rsqrt(jnp.maximum(deg, 1e-30)), 0.0)
    h0 = a[:, :d0]
    g1_out[...] = dinv * jnp.dot(h0, w1g[...], preferred_element_type=jnp.float32)
    dinv_out[...] = dinv


def _stage_c(h0aug, gcn_w1):
    n_pad, d0a = h0aug.shape
    d1 = gcn_w1.shape[1]
    bn = 1024
    return pl.pallas_call(
        _c_body,
        grid=(n_pad // bn,),
        in_specs=[
            pl.BlockSpec((bn, d0a), lambda i: (i, 0)),
            pl.BlockSpec(gcn_w1.shape, lambda i: (0, 0)),
        ],
        out_specs=[
            pl.BlockSpec((bn, d1), lambda i: (i, 0)),
            pl.BlockSpec((bn, 1), lambda i: (i, 0)),
        ],
        out_shape=[
            jax.ShapeDtypeStruct((n_pad, d1), jnp.float32),
            jax.ShapeDtypeStruct((n_pad, 1), jnp.float32),
        ],
    )(h0aug, gcn_w1)


def _d_body(t1, dinv_ref, b1g, w2g, g2_out):
    t = t1[0] + t1[1]
    dinv = dinv_ref[...]
    h1 = jnp.maximum(dinv * t + b1g[...], 0.0)
    g2_out[...] = dinv * jnp.dot(h1, w2g[...], preferred_element_type=jnp.float32)


def _stage_d(t1, dinv, gcn_b1, gcn_w2):
    n_pad = t1.shape[1]
    d1 = t1.shape[2]
    d2 = gcn_w2.shape[1]
    bn = 1024
    return pl.pallas_call(
        _d_body,
        grid=(n_pad // bn,),
        in_specs=[
            pl.BlockSpec((2, bn, d1), lambda i: (0, i, 0)),
            pl.BlockSpec((bn, 1), lambda i: (i, 0)),
            pl.BlockSpec((1, d1), lambda i: (0, 0)),
            pl.BlockSpec(gcn_w2.shape, lambda i: (0, 0)),
        ],
        out_specs=pl.BlockSpec((bn, d2), lambda i: (i, 0)),
        out_shape=jax.ShapeDtypeStruct((n_pad, d2), jnp.float32),
    )(t1, dinv, gcn_b1.reshape(1, -1), gcn_w2)


def _e_body(t2, dinv_ref, b2g, out):
    out[...] = jnp.maximum(dinv_ref[...] * (t2[0] + t2[1]) + b2g[...], 0.0)


def _stage_e(t2, dinv, gcn_b2):
    n_pad = t2.shape[1]
    d2 = t2.shape[2]
    bn = 1024
    return pl.pallas_call(
        _e_body,
        grid=(n_pad // bn,),
        in_specs=[
            pl.BlockSpec((2, bn, d2), lambda i: (0, i, 0)),
            pl.BlockSpec((bn, 1), lambda i: (i, 0)),
            pl.BlockSpec((1, d2), lambda i: (0, 0)),
        ],
        out_specs=pl.BlockSpec((bn, d2), lambda i: (i, 0)),
        out_shape=jax.ShapeDtypeStruct((n_pad, d2), jnp.float32),
    )(t2, dinv, gcn_b2.reshape(1, -1))


def kernel(x, edge_attr, edge_index, emb_table,
           phi_w0, phi_b0, phi_w1, phi_b1, phi_w2, phi_b2, phi_w3, phi_b3,
           gcn_w1, gcn_b1, gcn_w2, gcn_b2):
    n = x.shape[0]
    e = edge_index.shape[1]
    hid = phi_w0.shape[1]
    d0 = phi_w3.shape[1]
    d0a = d0 + 32
    n_pad = _ceil_to(n, 2048)
    e_pad = _ceil_to(e, 4096)

    row = edge_index[0]
    col = edge_index[1]
    ep = e_pad - e
    rowp = jnp.concatenate([row, jnp.zeros((ep,), jnp.int32)])
    colp = jnp.concatenate([col, jnp.full((ep,), n, jnp.int32)])
    eap = jnp.concatenate([edge_attr[:, 0], jnp.zeros((ep,), jnp.float32)])[:, None]

    xp = jnp.pad(x, ((0, n_pad - n), (0, 0)))
    nf16 = _prep_table(xp, emb_table, n_pad)

    gcol = jnp.take(nf16, colp, axis=0)
    grow = jnp.take(nf16, rowp, axis=0)

    nemb = emb_table.shape[1]
    nfd = 4 + nemb
    w0cat = jnp.zeros((32, hid), jnp.float32)
    w0cat = w0cat.at[0:nfd].set(phi_w0[0:nfd])
    w0cat = w0cat.at[16:16 + nfd].set(phi_w0[nfd:2 * nfd])
    w0e = phi_w0[2 * nfd:2 * nfd + 1]
    w3a = jnp.pad(phi_w3, ((0, 0), (0, 32)))
    b3a = jnp.pad(phi_b3, (0, 32))[None, :]

    msgaug = _mlp(gcol, grow, eap, w0cat, w0e, phi_b0[None, :],
                  phi_w1, phi_b1[None, :], phi_w2, phi_b2[None, :], w3a, b3a)

    h0aug = jax.ops.segment_sum(msgaug, colp, num_segments=n_pad)

    g1, dinv = _stage_c(h0aug, gcn_w1)

    t1h = jax.ops.segment_sum(eap * jnp.take(g1, rowp, axis=0), colp, num_segments=n_pad)
    t1 = jnp.stack([t1h, jnp.zeros_like(t1h)])

    g2 = _stage_d(t1, dinv, gcn_b1, gcn_w2)

    t2h = jax.ops.segment_sum(eap * jnp.take(g2, rowp, axis=0), colp, num_segments=n_pad)
    t2 = jnp.stack([t2h, jnp.zeros_like(t2h)])

    h2 = _stage_e(t2, dinv, gcn_b2)
    return h2[:n]
